# Initial kernel scaffold; baseline (speedup 1.0000x reference)
import jax
import jax.numpy as jnp
from jax.experimental import pallas as pl


def kernel(f0, emb, W1, b1, W2, b2):
    raise NotImplementedError("write your pallas kernel here")


# speedup vs baseline: 4.0837x; 4.0837x over previous
import functools
import math

import jax
import jax.numpy as jnp
from jax import lax
from jax.experimental import pallas as pl
from jax.experimental.pallas import tpu as pltpu
from jax.experimental.pallas import tpu_sc as plsc

N_F0_BINS = 256
V1_DIM = 768
HIDDEN_DIM = 512
F0_MIN = 50.0
F0_MAX = 1100.0

_MEL_MIN = 1127.0 * math.log(1.0 + F0_MIN / 700.0)
_MEL_MAX = 1127.0 * math.log(1.0 + F0_MAX / 700.0)

_NC = 2
_NS = 16
_NW = _NC * _NS


def _prep_body(f0_ref, emb_ref, w1_ref, b1_ref, w2_ref, b2_ref,
               bins_ref, table_ref):
    f0 = f0_ref[...]
    f0_mel = 1127.0 * jnp.log(1.0 + f0 / 700.0)
    f0_mel = jnp.where(
        f0_mel > 0.0,
        (f0_mel - _MEL_MIN) * (N_F0_BINS - 2) / (_MEL_MAX - _MEL_MIN) + 1.0,
        f0_mel,
    )
    f0_mel = jnp.where(f0_mel <= 1.0, 1.0, f0_mel)
    f0_mel = jnp.where(f0_mel > N_F0_BINS - 1, float(N_F0_BINS - 1), f0_mel)
    bins_ref[...] = (f0_mel + 0.5).astype(jnp.int32)

    h = jnp.dot(emb_ref[...], w1_ref[...], preferred_element_type=jnp.float32)
    h = h + b1_ref[...]
    h = 0.5 * h * (1.0 + lax.erf(h * (1.0 / math.sqrt(2.0))))
    t = jnp.dot(h, w2_ref[...], preferred_element_type=jnp.float32)
    table_ref[...] = t + b2_ref[...]


def _make_sc_gather(n_rows, d, chunk):
    n_per_w = n_rows // _NW
    n_chunks = n_per_w // chunk
    mesh = plsc.VectorSubcoreMesh(core_axis_name="c", subcore_axis_name="s")

    @functools.partial(
        pl.kernel,
        mesh=mesh,
        out_type=jax.ShapeDtypeStruct((n_rows, d), jnp.float32),
        scratch_types=[
            pltpu.VMEM((n_per_w,), jnp.int32),
            pltpu.VMEM((chunk, d), jnp.float32),
            pltpu.SemaphoreType.DMA,
        ],
    )
    def gather_kernel(table_hbm, bins_hbm, out_hbm, idx_v, rows_v, sem):
        wid = lax.axis_index("s") * _NC + lax.axis_index("c")
        base = wid * n_per_w
        pltpu.sync_copy(bins_hbm.at[pl.ds(base, n_per_w)], idx_v)

        def body(i, _):
            off = i * chunk
            idx_chunk = idx_v.at[pl.ds(off, chunk)]
            pltpu.async_copy(table_hbm.at[idx_chunk], rows_v, sem).wait()
            pltpu.sync_copy(rows_v, out_hbm.at[pl.ds(base + off, chunk)])
            return 0

        lax.fori_loop(0, n_chunks, body, 0)

    return gather_kernel


def kernel(f0, emb, W1, b1, W2, b2):
    B, T = f0.shape
    d = W2.shape[1]

    bins, table = pl.pallas_call(
        _prep_body,
        out_shape=(
            jax.ShapeDtypeStruct((B, T), jnp.int32),
            jax.ShapeDtypeStruct((N_F0_BINS, d), jnp.float32),
        ),
    )(f0, emb, W1, b1.reshape(1, -1), W2, b2.reshape(1, -1))

    n_rows = B * T
    out = _make_sc_gather(n_rows, d, chunk=64)(table, bins.reshape(n_rows))
    return out.reshape(B, T, d)


# speedup vs baseline: 4.1424x; 1.0144x over previous
import functools
import math

import jax
import jax.numpy as jnp
from jax import lax
from jax.experimental import pallas as pl
from jax.experimental.pallas import tpu as pltpu
from jax.experimental.pallas import tpu_sc as plsc

N_F0_BINS = 256
V1_DIM = 768
HIDDEN_DIM = 512
F0_MIN = 50.0
F0_MAX = 1100.0

_MEL_MIN = 1127.0 * math.log(1.0 + F0_MIN / 700.0)
_MEL_MAX = 1127.0 * math.log(1.0 + F0_MAX / 700.0)

_NC = 2
_NS = 16
_NW = _NC * _NS


def _prep_body(f0_ref, emb_ref, w1_ref, b1_ref, w2_ref, b2_ref,
               bins_ref, table_ref):
    f0 = f0_ref[...]
    f0_mel = 1127.0 * jnp.log(1.0 + f0 / 700.0)
    f0_mel = jnp.where(
        f0_mel > 0.0,
        (f0_mel - _MEL_MIN) * (N_F0_BINS - 2) / (_MEL_MAX - _MEL_MIN) + 1.0,
        f0_mel,
    )
    f0_mel = jnp.where(f0_mel <= 1.0, 1.0, f0_mel)
    f0_mel = jnp.where(f0_mel > N_F0_BINS - 1, float(N_F0_BINS - 1), f0_mel)
    bins_ref[...] = (f0_mel + 0.5).astype(jnp.int32)

    h = jnp.dot(emb_ref[...], w1_ref[...], preferred_element_type=jnp.float32)
    h = h + b1_ref[...]
    h = 0.5 * h * (1.0 + lax.erf(h * (1.0 / math.sqrt(2.0))))
    t = jnp.dot(h, w2_ref[...], preferred_element_type=jnp.float32)
    table_ref[...] = t + b2_ref[...]


_NBUF = 4


def _make_sc_gather(n_rows, d, chunk):
    n_per_w = n_rows // _NW
    n_chunks = n_per_w // chunk
    assert n_chunks % _NBUF == 0
    mesh = plsc.VectorSubcoreMesh(core_axis_name="c", subcore_axis_name="s")

    @functools.partial(
        pl.kernel,
        mesh=mesh,
        out_type=jax.ShapeDtypeStruct((n_rows, d), jnp.float32),
        scratch_types=[
            pltpu.VMEM((n_per_w,), jnp.int32),
            pltpu.VMEM((_NBUF, chunk, d), jnp.float32),
        ]
        + [pltpu.SemaphoreType.DMA] * (2 * _NBUF),
    )
    def gather_kernel(table_hbm, bins_hbm, out_hbm, idx_v, rows_v, *sems):
        gsems, ssems = sems[:_NBUF], sems[_NBUF:]
        wid = lax.axis_index("s") * _NC + lax.axis_index("c")
        base = wid * n_per_w
        pltpu.sync_copy(bins_hbm.at[pl.ds(base, n_per_w)], idx_v)

        def gather_chunk(off, b):
            return pltpu.make_async_copy(
                table_hbm.at[idx_v.at[pl.ds(off, chunk)]],
                rows_v.at[b], gsems[b])

        def store_chunk(off, b):
            return pltpu.make_async_copy(
                rows_v.at[b], out_hbm.at[pl.ds(base + off, chunk)], ssems[b])

        for b in range(_NBUF):
            gather_chunk(b * chunk, b).start()

        def body(j, _):
            for b in range(_NBUF):
                i = j * _NBUF + b
                off = i * chunk
                gather_chunk(off, b).wait()
                store_chunk(off, b).start()
                store_chunk(off, b).wait()

                @pl.when(j + 1 < n_chunks // _NBUF)
                def _():
                    gather_chunk(off + _NBUF * chunk, b).start()

            return 0

        lax.fori_loop(0, n_chunks // _NBUF, body, 0)

    return gather_kernel


def kernel(f0, emb, W1, b1, W2, b2):
    B, T = f0.shape
    d = W2.shape[1]

    bins, table = pl.pallas_call(
        _prep_body,
        out_shape=(
            jax.ShapeDtypeStruct((B, T), jnp.int32),
            jax.ShapeDtypeStruct((N_F0_BINS, d), jnp.float32),
        ),
    )(f0, emb, W1, b1.reshape(1, -1), W2, b2.reshape(1, -1))

    n_rows = B * T
    out = _make_sc_gather(n_rows, d, chunk=32)(table, bins.reshape(n_rows))
    return out.reshape(B, T, d)
